# baseline (speedup 1.0000x reference)
import functools

import jax
import jax.numpy as jnp
from jax import lax
from jax.experimental import pallas as pl
from jax.experimental.pallas import tpu as pltpu
from jax.experimental.pallas import tpu_sc as plsc


@functools.lru_cache(maxsize=None)
def _build_gather(B, V, D):
    info = plsc.get_sparse_core_info()
    nc, ns = info.num_cores, info.num_subcores
    nw = nc * ns
    b_per_w = B // nw
    chunk = 128
    n_chunk = b_per_w // chunk
    mesh = plsc.VectorSubcoreMesh(core_axis_name="c", subcore_axis_name="s")

    @functools.partial(
        pl.kernel,
        mesh=mesh,
        out_type=jax.ShapeDtypeStruct((B, D), jnp.float32),
        scratch_types=[
            pltpu.VMEM((n_chunk, chunk), jnp.int32),
            pltpu.VMEM((b_per_w, D), jnp.float32),
            pltpu.SemaphoreType.DMA,
        ],
        compiler_params=pltpu.CompilerParams(use_tc_tiling_on_sc=False),
    )
    def gather(idx_hbm, table_hbm, out_hbm, idx_v, rows_v, sem):
        wid = lax.axis_index("s") * nc + lax.axis_index("c")
        base = wid * b_per_w
        pltpu.sync_copy(idx_hbm.at[wid], idx_v)
        copies = [
            pltpu.async_copy(
                table_hbm.at[idx_v.at[j]],
                rows_v.at[pl.ds(j * chunk, chunk)],
                sem,
            )
            for j in range(n_chunk)
        ]
        for c in copies:
            c.wait()
        pltpu.sync_copy(rows_v, out_hbm.at[pl.ds(base, b_per_w)])

    return gather, nw, n_chunk, chunk


def kernel(inputs, embeddings):
    B = inputs.shape[0]
    V, D = embeddings.shape
    gather, nw, n_chunk, chunk = _build_gather(B, V, D)
    idx = inputs.astype(jnp.int32).reshape(nw, n_chunk, chunk)
    out = gather(idx, embeddings)
    return out[None]


# speedup vs baseline: 1.0080x; 1.0080x over previous
import functools

import jax
import jax.numpy as jnp
from jax import lax
from jax.experimental import pallas as pl
from jax.experimental.pallas import tpu as pltpu
from jax.experimental.pallas import tpu_sc as plsc


@functools.lru_cache(maxsize=None)
def _build_gather(B, V, D):
    info = plsc.get_sparse_core_info()
    nc, ns = info.num_cores, info.num_subcores
    nw = nc * ns
    b_per_w = B // nw
    chunk = 128
    n_chunk = b_per_w // chunk
    mesh = plsc.VectorSubcoreMesh(core_axis_name="c", subcore_axis_name="s")

    @functools.partial(
        pl.kernel,
        mesh=mesh,
        out_type=jax.ShapeDtypeStruct((B, D), jnp.float32),
        scratch_types=[
            pltpu.VMEM((b_per_w,), jnp.int32),
            pltpu.VMEM((b_per_w, D), jnp.float32),
            pltpu.SemaphoreType.DMA,
        ],
        compiler_params=pltpu.CompilerParams(use_tc_tiling_on_sc=False),
    )
    def gather(idx_hbm, table_hbm, out_hbm, idx_v, rows_v, sem):
        wid = lax.axis_index("s") * nc + lax.axis_index("c")
        base = wid * b_per_w
        pltpu.sync_copy(idx_hbm.at[pl.ds(base, b_per_w)], idx_v)
        copies = [
            pltpu.async_copy(
                table_hbm.at[idx_v.at[pl.ds(j * chunk, chunk)]],
                rows_v.at[pl.ds(j * chunk, chunk)],
                sem,
            )
            for j in range(n_chunk)
        ]
        for c in copies:
            c.wait()
        pltpu.sync_copy(rows_v, out_hbm.at[pl.ds(base, b_per_w)])

    return gather


def kernel(inputs, embeddings):
    B = inputs.shape[0]
    V, D = embeddings.shape
    gather = _build_gather(B, V, D)
    out = gather(inputs.astype(jnp.int32), embeddings)
    return out[None]


# speedup vs baseline: 1.1681x; 1.1588x over previous
import functools

import jax
import jax.numpy as jnp
from jax import lax
from jax.experimental import pallas as pl
from jax.experimental.pallas import tpu as pltpu
from jax.experimental.pallas import tpu_sc as plsc


@functools.lru_cache(maxsize=None)
def _build_gather(B, V, D):
    info = plsc.get_sparse_core_info()
    nc, ns = info.num_cores, info.num_subcores
    nw = nc * ns
    b_per_w = B // nw
    group = 16
    n_group = b_per_w // group
    mesh = plsc.VectorSubcoreMesh(core_axis_name="c", subcore_axis_name="s")

    @functools.partial(
        pl.kernel,
        mesh=mesh,
        out_type=jax.ShapeDtypeStruct((B, D), jnp.float32),
        scratch_types=[
            pltpu.VMEM((b_per_w,), jnp.int32),
            pltpu.VMEM((b_per_w, D), jnp.float32),
            pltpu.SemaphoreType.DMA,
        ],
    )
    def gather(idx_hbm, table_hbm, out_hbm, idx_v, rows_v, sem):
        wid = lax.axis_index("s") * nc + lax.axis_index("c")
        base = wid * b_per_w
        pltpu.sync_copy(idx_hbm.at[pl.ds(base, b_per_w)], idx_v)

        def body(g, _):
            v = idx_v[pl.ds(g * group, group)]
            for j in range(group):
                i = g * group + j
                r = v[j]
                pltpu.async_copy(
                    table_hbm.at[pl.ds(r, 1)],
                    rows_v.at[pl.ds(i, 1)],
                    sem,
                )
            for j in range(group):
                pltpu.make_async_copy(
                    table_hbm.at[pl.ds(0, 1)],
                    rows_v.at[pl.ds(g * group + j, 1)],
                    sem,
                ).wait()
            return _

        lax.fori_loop(0, n_group, body, 0)
        pltpu.sync_copy(rows_v, out_hbm.at[pl.ds(base, b_per_w)])

    return gather


def kernel(inputs, embeddings):
    B = inputs.shape[0]
    V, D = embeddings.shape
    gather = _build_gather(B, V, D)
    out = gather(inputs.astype(jnp.int32), embeddings)
    return out[None]


# speedup vs baseline: 1.9432x; 1.6635x over previous
import functools

import jax
import jax.numpy as jnp
from jax import lax
from jax.experimental import pallas as pl
from jax.experimental.pallas import tpu as pltpu
from jax.experimental.pallas import tpu_sc as plsc


@functools.lru_cache(maxsize=None)
def _build_gather(B, V, D):
    info = plsc.get_sparse_core_info()
    nc, ns, nl = info.num_cores, info.num_subcores, info.num_lanes
    nw = nc * ns
    d_per_w = D // nw
    half = B // 2
    mesh = plsc.VectorSubcoreMesh(core_axis_name="c", subcore_axis_name="s")

    @functools.partial(
        pl.kernel,
        mesh=mesh,
        out_type=jax.ShapeDtypeStruct((D, B), jnp.float32),
        scratch_types=[
            pltpu.VMEM((V,), jnp.float32),
            pltpu.VMEM((half,), jnp.int32),
            pltpu.VMEM((B,), jnp.float32),
            pltpu.SemaphoreType.DMA,
        ],
        compiler_params=pltpu.CompilerParams(needs_layout_passes=False),
    )
    def gather(idx_hbm, table_hbm, out_hbm, row_v, idx_v, obuf_v, sem):
        wid = lax.axis_index("s") * nc + lax.axis_index("c")

        for dd in range(d_per_w):
            d = wid * d_per_w + dd
            pltpu.sync_copy(table_hbm.at[d], row_v)
            for h in range(2):
                pltpu.sync_copy(idx_hbm.at[pl.ds(h * half, half)], idx_v)

                def body(k, _):
                    iv = idx_v[pl.ds(k * nl, nl)]
                    obuf_v[pl.ds(h * half + k * nl, nl)] = plsc.load_gather(
                        row_v, [iv]
                    )
                    return _

                lax.fori_loop(0, half // nl, body, 0)
            pltpu.sync_copy(obuf_v, out_hbm.at[d])

    def run(inputs, embeddings):
        p = jnp.transpose(embeddings)
        out_t = gather(inputs.astype(jnp.int32), p)
        return jnp.transpose(out_t)[None]

    return run


def kernel(inputs, embeddings):
    B = inputs.shape[0]
    V, D = embeddings.shape
    return _build_gather(B, V, D)(inputs, embeddings)


# speedup vs baseline: 2.2261x; 1.1456x over previous
import functools

import jax
import jax.numpy as jnp
from jax import lax
from jax.experimental import pallas as pl
from jax.experimental.pallas import tpu as pltpu
from jax.experimental.pallas import tpu_sc as plsc

_UNROLL = 8


@functools.lru_cache(maxsize=None)
def _build_gather(B, V, D):
    info = plsc.get_sparse_core_info()
    nc, ns, nl = info.num_cores, info.num_subcores, info.num_lanes
    nw = nc * ns
    d_per_w = D // nw
    half = B // 2
    step = nl * _UNROLL
    mesh = plsc.VectorSubcoreMesh(core_axis_name="c", subcore_axis_name="s")

    @functools.partial(
        pl.kernel,
        mesh=mesh,
        out_type=jax.ShapeDtypeStruct((D, B), jnp.float32),
        scratch_types=[
            pltpu.VMEM((V,), jnp.float32),
            pltpu.VMEM((half,), jnp.int32),
            pltpu.VMEM((B,), jnp.float32),
            pltpu.SemaphoreType.DMA,
            pltpu.SemaphoreType.DMA,
        ],
        compiler_params=pltpu.CompilerParams(needs_layout_passes=False),
    )
    def gather(idx_hbm, table_hbm, out_hbm, row_v, idx_v, obuf_v, rsem, wsem):
        wid = lax.axis_index("s") * nc + lax.axis_index("c")

        for dd in range(d_per_w):
            d = wid * d_per_w + dd
            pltpu.async_copy(table_hbm.at[d], row_v, rsem)
            if dd > 0:
                pltpu.make_async_copy(obuf_v, out_hbm.at[d - 1], wsem).wait()
            pltpu.make_async_copy(table_hbm.at[d], row_v, rsem).wait()
            for h in range(2):
                pltpu.sync_copy(idx_hbm.at[pl.ds(h * half, half)], idx_v)

                def body(k, _):
                    base = k * step
                    for u in range(_UNROLL):
                        o = base + u * nl
                        iv = idx_v[pl.ds(o, nl)]
                        obuf_v[pl.ds(h * half + o, nl)] = plsc.load_gather(
                            row_v, [iv]
                        )
                    return _

                lax.fori_loop(0, half // step, body, 0)
            pltpu.async_copy(obuf_v, out_hbm.at[d], wsem)
        d_last = wid * d_per_w + d_per_w - 1
        pltpu.make_async_copy(obuf_v, out_hbm.at[d_last], wsem).wait()

    def run(inputs, embeddings):
        p = jnp.transpose(embeddings)
        out_t = gather(inputs.astype(jnp.int32), p)
        return jnp.transpose(out_t)[None]

    return run


def kernel(inputs, embeddings):
    B = inputs.shape[0]
    V, D = embeddings.shape
    return _build_gather(B, V, D)(inputs, embeddings)
